# baseline (speedup 1.0000x reference)
import functools

import jax
import jax.numpy as jnp
from jax import lax
from jax.experimental import pallas as pl
from jax.experimental.pallas import tpu as pltpu
from jax.experimental.pallas import tpu_sc as plsc

_NC = 2
_NS = 16
_NW = _NC * _NS

_GB = 8


@functools.partial(jax.jit, static_argnames=("batch", "hist", "dim"))
def _sc_gather(table, word_ids, *, batch, hist, dim):
    per_w = batch // _NW
    groups = per_w // _GB

    mesh = plsc.VectorSubcoreMesh(core_axis_name="c", subcore_axis_name="s")

    @functools.partial(
        pl.kernel,
        mesh=mesh,
        compiler_params=pltpu.CompilerParams(use_tc_tiling_on_sc=False),
        out_type=jax.ShapeDtypeStruct((batch, hist, dim), jnp.float32),
        scratch_types=[
            pltpu.VMEM((_GB, hist), jnp.int32),
            pltpu.VMEM((_GB, hist, dim), jnp.float32),
            pltpu.SemaphoreType.DMA,
        ],
    )
    def k(table_hbm, ids_hbm, out_hbm, idx_v, rows_v, gsem):
        wid = lax.axis_index("s") * _NC + lax.axis_index("c")
        w_base = wid * per_w

        def body(g, _):
            b0 = w_base + g * _GB
            pltpu.sync_copy(ids_hbm.at[pl.ds(b0, _GB)], idx_v)
            cps = []
            for i in range(_GB):
                cps.append(
                    pltpu.async_copy(
                        table_hbm.at[idx_v.at[i]], rows_v.at[i], gsem
                    )
                )
            for cp in cps:
                cp.wait()
            pltpu.sync_copy(rows_v, out_hbm.at[pl.ds(b0, _GB)])
            return 0

        lax.fori_loop(0, groups, body, 0)

    return k(table, word_ids)


def kernel(word_ids, table):
    b, h = word_ids.shape
    v, d = table.shape
    return _sc_gather(table, word_ids, batch=b, hist=h, dim=d)
